# baseline (speedup 1.0000x reference)
import jax
import jax.numpy as jnp
from jax import lax
from jax.experimental import pallas as pl
from jax.experimental.pallas import tpu as pltpu
from jax.experimental.pallas import tpu_sc as plsc

_B, _L, _V, _D = 4096, 50, 100000, 300
_DP = 384
_DT = 304
_NC, _NS = 2, 16
_NW = _NC * _NS
_LANES = 16
_BPW = _B // _NW
_FBLOCKS = ((0, 64), (64, 64), (128, 64), (192, 64), (256, 48))


def _sc_body(ids_hbm, len_hbm, table_hbm, emb_hbm, mask_hbm,
             idx_v, len_v, mbuf, buf, tbuf, sem):
    wid = lax.axis_index("s") * _NC + lax.axis_index("c")
    b0 = wid * _BPW

    pltpu.sync_copy(len_hbm.at[pl.ds(b0, _BPW)], len_v)

    def l_body(l, carry):
        pltpu.sync_copy(ids_hbm.at[l, pl.ds(b0, _BPW)], idx_v)
        pltpu.async_copy(
            table_hbm.at[idx_v], buf.at[:, pl.ds(0, _DP)], sem).wait()

        for j in range(_BPW // _LANES):
            lens16 = len_v[pl.ds(j * _LANES, _LANES)]
            mbuf[pl.ds(j * _LANES, _LANES)] = (l < lens16).astype(jnp.float32)
        pltpu.sync_copy(mbuf, mask_hbm.at[l, pl.ds(b0, _BPW)])

        for fb0, fn in _FBLOCKS:
            def f_body(f, c2, fb0=fb0):
                col = jnp.full((_LANES,), fb0 + f, jnp.int32)
                for j in range(_BPW // _LANES):
                    rows = j * _LANES + lax.iota(jnp.int32, _LANES)
                    tbuf[f, pl.ds(j * _LANES, _LANES)] = \
                        plsc.load_gather(buf, [rows, col])
                return c2

            lax.fori_loop(0, fn, f_body, 0, unroll=4)
            pltpu.sync_copy(tbuf.at[pl.ds(0, fn)],
                            emb_hbm.at[l, pl.ds(fb0, fn), pl.ds(b0, _BPW)])
        return carry

    lax.fori_loop(0, _L, l_body, 0)


@jax.jit
def _sc_call(ids_t, lens, table_pad):
    mesh = plsc.VectorSubcoreMesh(
        core_axis_name="c", subcore_axis_name="s",
        num_cores=_NC, num_subcores=_NS)
    fn = pl.kernel(
        _sc_body,
        out_type=[
            jax.ShapeDtypeStruct((_L, _DT, _B), jnp.float32),
            jax.ShapeDtypeStruct((_L, _B), jnp.float32),
        ],
        mesh=mesh,
        scratch_types=[
            pltpu.VMEM((_BPW,), jnp.int32),
            pltpu.VMEM((_BPW,), jnp.int32),
            pltpu.VMEM((_BPW,), jnp.float32),
            pltpu.VMEM((_BPW, _DP + 1), jnp.float32),
            pltpu.VMEM((64, _BPW), jnp.float32),
            pltpu.SemaphoreType.DMA,
        ],
        compiler_params=pltpu.CompilerParams(
            needs_layout_passes=False, use_tc_tiling_on_sc=True),
    )
    return fn(ids_t, lens, table_pad)


def kernel(token_ids, lengths, fasttext_table):
    assert token_ids.shape == (_B, _L) and fasttext_table.shape == (_V, _D)
    lens = lengths.astype(jnp.int32)
    ids_t = token_ids.T.astype(jnp.int32)
    valid = jnp.arange(_L, dtype=jnp.int32)[:, None] < lens[None, :]
    ids_m = jnp.where(valid, ids_t, _V)
    table_pad = jnp.pad(fasttext_table.astype(jnp.float32),
                        ((0, 1), (0, _DP - _D)))
    emb_phys, mask_phys = _sc_call(ids_m, lens, table_pad)
    emb = jnp.transpose(emb_phys, (2, 0, 1))[:, :, :_D]
    mask = jnp.transpose(mask_phys, (1, 0))
    return emb, mask


# speedup vs baseline: 1.0203x; 1.0203x over previous
import jax
import jax.numpy as jnp
from jax import lax
from jax.experimental import pallas as pl
from jax.experimental.pallas import tpu as pltpu
from jax.experimental.pallas import tpu_sc as plsc

_B, _L, _V, _D = 4096, 50, 100000, 300
_DP = 384
_DT = 304
_NC, _NS = 2, 16
_NW = _NC * _NS
_LANES = 16
_BPW = _B // _NW
_FBLOCKS = ((0, 64), (64, 64), (128, 64), (192, 64), (256, 48))


def _sc_body(ids_hbm, len_hbm, table_hbm, emb_hbm, mask_hbm,
             idx_v, len_v, mbuf, buf, tbuf, sem):
    wid = lax.axis_index("s") * _NC + lax.axis_index("c")
    b0 = wid * _BPW

    pltpu.sync_copy(len_hbm.at[pl.ds(b0, _BPW)], len_v)

    def l_body(l, carry):
        pltpu.sync_copy(ids_hbm.at[l, pl.ds(b0, _BPW)], idx_v)
        pltpu.async_copy(
            table_hbm.at[idx_v], buf.at[:, pl.ds(0, _DP)], sem).wait()

        for j in range(_BPW // _LANES):
            lens16 = len_v[pl.ds(j * _LANES, _LANES)]
            mbuf[pl.ds(j * _LANES, _LANES)] = (l < lens16).astype(jnp.float32)
        pltpu.sync_copy(mbuf, mask_hbm.at[l, pl.ds(b0, _BPW)])

        for fb0, fn in _FBLOCKS:
            @plsc.parallel_loop(0, fn, 1, unroll=8)
            def _transpose_f(f, fb0=fb0):
                col = jnp.full((_LANES,), fb0 + f, jnp.int32)
                for j in range(_BPW // _LANES):
                    rows = j * _LANES + lax.iota(jnp.int32, _LANES)
                    tbuf[f, pl.ds(j * _LANES, _LANES)] = \
                        plsc.load_gather(buf, [rows, col])
            pltpu.sync_copy(tbuf.at[pl.ds(0, fn)],
                            emb_hbm.at[l, pl.ds(fb0, fn), pl.ds(b0, _BPW)])
        return carry

    lax.fori_loop(0, _L, l_body, 0)


@jax.jit
def _sc_call(ids_t, lens, table_pad):
    mesh = plsc.VectorSubcoreMesh(
        core_axis_name="c", subcore_axis_name="s",
        num_cores=_NC, num_subcores=_NS)
    fn = pl.kernel(
        _sc_body,
        out_type=[
            jax.ShapeDtypeStruct((_L, _DT, _B), jnp.float32),
            jax.ShapeDtypeStruct((_L, _B), jnp.float32),
        ],
        mesh=mesh,
        scratch_types=[
            pltpu.VMEM((_BPW,), jnp.int32),
            pltpu.VMEM((_BPW,), jnp.int32),
            pltpu.VMEM((_BPW,), jnp.float32),
            pltpu.VMEM((_BPW, _DP + 1), jnp.float32),
            pltpu.VMEM((64, _BPW), jnp.float32),
            pltpu.SemaphoreType.DMA,
        ],
        compiler_params=pltpu.CompilerParams(
            needs_layout_passes=False, use_tc_tiling_on_sc=True),
    )
    return fn(ids_t, lens, table_pad)


def kernel(token_ids, lengths, fasttext_table):
    assert token_ids.shape == (_B, _L) and fasttext_table.shape == (_V, _D)
    lens = lengths.astype(jnp.int32)
    ids_t = token_ids.T.astype(jnp.int32)
    valid = jnp.arange(_L, dtype=jnp.int32)[:, None] < lens[None, :]
    ids_m = jnp.where(valid, ids_t, _V)
    table_pad = jnp.pad(fasttext_table.astype(jnp.float32),
                        ((0, 1), (0, _DP - _D)))
    emb_phys, mask_phys = _sc_call(ids_m, lens, table_pad)
    emb = jnp.transpose(emb_phys, (2, 0, 1))[:, :, :_D]
    mask = jnp.transpose(mask_phys, (1, 0))
    return emb, mask
